# baseline (speedup 1.0000x reference)
import jax
import jax.numpy as jnp
import numpy as np
from jax import lax
from jax.experimental import pallas as pl
from jax.experimental.pallas import tpu as pltpu

_EPS = 0.001
_GRIDSIZE = 100
_MAX_ITER = 3
_INT_MIN = np.int32(-(2**31))


def _body(x_ref, k_ref, k0_ref, grid_ref, out_ref):
    x = x_ref[:]
    R, D = x.shape
    kvec = k_ref[:]
    kf = kvec.astype(jnp.float32)
    k0 = k0_ref[0]

    b = lax.bitcast_convert_type(x, jnp.int32)
    ukey = jnp.where(b >= 0, b ^ _INT_MIN, ~b)

    p0 = jnp.zeros((R, 1), jnp.int32)
    p1 = jnp.zeros((R, 1), jnp.int32)
    r0 = kvec
    r1 = jnp.broadcast_to(k0, (R, 1))
    ones_col = jnp.ones((D, 1), jnp.float32)
    ukey_u = lax.bitcast_convert_type(ukey, jnp.uint32)
    for i in range(31, -1, -1):
        bit = _INT_MIN if i == 31 else np.int32(1 << i)
        m0 = jnp.where(ukey_u >= lax.bitcast_convert_type(p0 | bit,
                                                          jnp.uint32),
                       1.0, 0.0)
        m1 = (ukey_u >= lax.bitcast_convert_type(p1 | bit,
                                                 jnp.uint32)).astype(jnp.int32)
        g0 = jnp.dot(m0, ones_col,
                     preferred_element_type=jnp.float32).astype(jnp.int32)
        g1 = jnp.sum(m1, axis=1, keepdims=True)
        c0 = g0 - (kvec - r0)
        c1 = g1 - (k0 - r1)
        tb0 = (c0 >= r0).astype(jnp.int32)
        tb1 = (c1 >= r1).astype(jnp.int32)
        p0 = p0 | (bit & (-tb0))
        p1 = p1 | (bit & (-tb1))
        r0 = r0 - c0 * (1 - tb0)
        r1 = r1 - c1 * (1 - tb1)
    kb = jnp.where(p0 < 0, p0 ^ _INT_MIN, ~p0)
    xk = lax.bitcast_convert_type(kb, jnp.float32)
    qb = jnp.where(p1 < 0, p1 ^ _INT_MIN, ~p1)
    xq = lax.bitcast_convert_type(qb, jnp.float32)

    nulow = -xk
    nuup = nulow + 1.0
    grid01 = grid_ref[:]

    gio = lax.broadcasted_iota(jnp.int32, (R, _GRIDSIZE), 1)

    def _round(args):
        nulow, nuup = args
        delta = nuup - nulow
        mask = delta > _EPS

        lo = jnp.zeros((R, 1), jnp.int32)
        hi = jnp.full((R, 1), _GRIDSIZE, jnp.int32)
        for _bs in range(7):
            mid = (lo + hi) // 2
            g01m = jnp.sum(jnp.where(gio == mid, grid01, 0.0), axis=1,
                           keepdims=True)
            nug = g01m * delta + nulow
            s = jnp.sum(jnp.clip(x + nug, 0.0, 1.0), axis=1, keepdims=True)
            below = (s < kf).astype(jnp.int32)
            lo = lo + (mid + 1 - lo) * below
            hi = hi + (mid - hi) * (1 - below)
        cnt = lo
        upix = jnp.clip(cnt, 1, _GRIDSIZE - 1)
        nug_all = grid01 * delta + nulow
        new_up = jnp.sum(jnp.where(gio == upix, nug_all, 0.0), axis=1,
                         keepdims=True)
        new_lo = jnp.sum(jnp.where(gio == upix - 1, nug_all, 0.0), axis=1,
                         keepdims=True)
        return (jnp.where(mask, new_lo, nulow), jnp.where(mask, new_up, nuup))

    for _ in range(_MAX_ITER):
        pred = jnp.any((nuup - nulow) > _EPS)
        nulow, nuup = lax.cond(pred, _round, lambda a: a, (nulow, nuup))

    nu = (nulow + nuup) / 2.0
    soft = jnp.clip(x + nu, 0.0, 1.0)

    v = jnp.clip(xq + nu, 0.0, 1.0)
    gt = (soft > v).astype(jnp.int32)
    eq = (soft == v).astype(jnp.int32)
    cnt_gt = jnp.sum(gt, axis=1, keepdims=True)
    rrem = k0 - cnt_gt
    pc = eq
    s = 1
    while s < D:
        pc = pc + jnp.concatenate(
            [jnp.zeros((R, s), jnp.int32), pc[:, : D - s]], axis=1)
        s *= 2
    take = (pc <= rrem).astype(jnp.int32)
    hardf = (gt | (eq & take)).astype(jnp.float32)
    out_ref[:] = (hardf - soft) + soft


def _run(x, k2, k0, grid01, interpret=False):
    R, D = x.shape
    RB = 64
    return pl.pallas_call(
        _body,
        grid=(R // RB,),
        in_specs=[
            pl.BlockSpec((RB, D), lambda i: (i, 0)),
            pl.BlockSpec((RB, 1), lambda i: (i, 0)),
            pl.BlockSpec(memory_space=pltpu.SMEM),
            pl.BlockSpec((1, _GRIDSIZE), lambda i: (0, 0)),
        ],
        out_specs=pl.BlockSpec((RB, D), lambda i: (i, 0)),
        out_shape=jax.ShapeDtypeStruct((R, D), x.dtype),
        interpret=interpret,
    )(x, k2, k0, grid01)


@jax.jit
def kernel(input, k):
    x = input
    R, D = x.shape
    k2 = k.reshape(R, 1)
    k0 = k[:1]
    grid01 = jnp.linspace(0.0, 1.0, _GRIDSIZE, dtype=x.dtype).reshape(
        1, _GRIDSIZE)
    return _run(x, k2, k0, grid01)


# speedup vs baseline: 1.1423x; 1.1423x over previous
import jax
import jax.numpy as jnp
import numpy as np
from jax import lax
from jax.experimental import pallas as pl
from jax.experimental.pallas import tpu as pltpu

_EPS = 0.001
_GRIDSIZE = 100
_MAX_ITER = 3
_INT_MIN = np.int32(-(2**31))


def _body(x_ref, k_ref, k0_ref, grid_ref, out_ref):
    x = x_ref[:]
    R, D = x.shape
    kvec = k_ref[:]
    kf = kvec.astype(jnp.float32)
    k0 = k0_ref[0]

    b = lax.bitcast_convert_type(x, jnp.int32)
    ukey = jnp.where(b >= 0, b ^ _INT_MIN, ~b)

    p0 = jnp.zeros((R, 1), jnp.int32)
    p1 = jnp.zeros((R, 1), jnp.int32)
    r0 = kvec
    r1 = jnp.broadcast_to(k0, (R, 1))
    ones_col = jnp.ones((D, 1), jnp.float32)
    ukey_u = lax.bitcast_convert_type(ukey, jnp.uint32)
    for i in range(31, -1, -1):
        bit = _INT_MIN if i == 31 else np.int32(1 << i)
        m0 = jnp.where(ukey_u >= lax.bitcast_convert_type(p0 | bit,
                                                          jnp.uint32),
                       1.0, 0.0)
        m1 = jnp.where(ukey_u >= lax.bitcast_convert_type(p1 | bit,
                                                          jnp.uint32),
                       1.0, 0.0)
        g0 = jnp.dot(m0, ones_col,
                     preferred_element_type=jnp.float32).astype(jnp.int32)
        g1 = jnp.dot(m1, ones_col,
                     preferred_element_type=jnp.float32).astype(jnp.int32)
        c0 = g0 - (kvec - r0)
        c1 = g1 - (k0 - r1)
        tb0 = (c0 >= r0).astype(jnp.int32)
        tb1 = (c1 >= r1).astype(jnp.int32)
        p0 = p0 | (bit & (-tb0))
        p1 = p1 | (bit & (-tb1))
        r0 = r0 - c0 * (1 - tb0)
        r1 = r1 - c1 * (1 - tb1)
    kb = jnp.where(p0 < 0, p0 ^ _INT_MIN, ~p0)
    xk = lax.bitcast_convert_type(kb, jnp.float32)
    qb = jnp.where(p1 < 0, p1 ^ _INT_MIN, ~p1)
    xq = lax.bitcast_convert_type(qb, jnp.float32)

    nulow = -xk
    nuup = nulow + 1.0
    grid01 = grid_ref[:]

    gio = lax.broadcasted_iota(jnp.int32, (R, _GRIDSIZE), 1)

    def _round(args):
        nulow, nuup = args
        delta = nuup - nulow
        mask = delta > _EPS

        lo = jnp.zeros((R, 1), jnp.int32)
        hi = jnp.full((R, 1), _GRIDSIZE, jnp.int32)
        for _bs in range(7):
            mid = (lo + hi) // 2
            g01m = jnp.sum(jnp.where(gio == mid, grid01, 0.0), axis=1,
                           keepdims=True)
            nug = g01m * delta + nulow
            s = jnp.sum(jnp.clip(x + nug, 0.0, 1.0), axis=1, keepdims=True)
            below = (s < kf).astype(jnp.int32)
            lo = lo + (mid + 1 - lo) * below
            hi = hi + (mid - hi) * (1 - below)
        cnt = lo
        upix = jnp.clip(cnt, 1, _GRIDSIZE - 1)
        nug_all = grid01 * delta + nulow
        new_up = jnp.sum(jnp.where(gio == upix, nug_all, 0.0), axis=1,
                         keepdims=True)
        new_lo = jnp.sum(jnp.where(gio == upix - 1, nug_all, 0.0), axis=1,
                         keepdims=True)
        return (jnp.where(mask, new_lo, nulow), jnp.where(mask, new_up, nuup))

    for _ in range(_MAX_ITER):
        pred = jnp.any((nuup - nulow) > _EPS)
        nulow, nuup = lax.cond(pred, _round, lambda a: a, (nulow, nuup))

    nu = (nulow + nuup) / 2.0
    soft = jnp.clip(x + nu, 0.0, 1.0)

    v = jnp.clip(xq + nu, 0.0, 1.0)
    gt = (soft > v).astype(jnp.int32)
    eq = (soft == v).astype(jnp.int32)
    cnt_gt = jnp.sum(gt, axis=1, keepdims=True)
    rrem = k0 - cnt_gt
    pc = eq
    s = 1
    while s < D:
        pc = pc + jnp.concatenate(
            [jnp.zeros((R, s), jnp.int32), pc[:, : D - s]], axis=1)
        s *= 2
    take = (pc <= rrem).astype(jnp.int32)
    hardf = (gt | (eq & take)).astype(jnp.float32)
    out_ref[:] = (hardf - soft) + soft


def _run(x, k2, k0, grid01, interpret=False):
    R, D = x.shape
    RB = 64
    return pl.pallas_call(
        _body,
        grid=(R // RB,),
        in_specs=[
            pl.BlockSpec((RB, D), lambda i: (i, 0)),
            pl.BlockSpec((RB, 1), lambda i: (i, 0)),
            pl.BlockSpec(memory_space=pltpu.SMEM),
            pl.BlockSpec((1, _GRIDSIZE), lambda i: (0, 0)),
        ],
        out_specs=pl.BlockSpec((RB, D), lambda i: (i, 0)),
        out_shape=jax.ShapeDtypeStruct((R, D), x.dtype),
        interpret=interpret,
    )(x, k2, k0, grid01)


@jax.jit
def kernel(input, k):
    x = input
    R, D = x.shape
    k2 = k.reshape(R, 1)
    k0 = k[:1]
    grid01 = jnp.linspace(0.0, 1.0, _GRIDSIZE, dtype=x.dtype).reshape(
        1, _GRIDSIZE)
    return _run(x, k2, k0, grid01)
